# baseline (speedup 1.0000x reference)
import functools

import jax
import jax.numpy as jnp
from jax import lax
from jax.experimental import pallas as pl
from jax.experimental.pallas import tpu as pltpu
from jax.experimental.pallas import tpu_sc as plsc

N = 100000
D = 128
S = 512
C = 128
NFULL = N // C
REM = N - NFULL * C
NW = 32
CPW = 25
RPT = S // 16
W = 128
CNT_W = 16


def _sc_pool(x, batch):
    mesh = plsc.VectorSubcoreMesh(core_axis_name="c", subcore_axis_name="s")

    @functools.partial(
        pl.kernel,
        mesh=mesh,
        out_type=[
            jax.ShapeDtypeStruct((2 * S, D), jnp.float32),
            jax.ShapeDtypeStruct((2 * S, D), jnp.float32),
        ],
        scratch_types=[
            pltpu.VMEM((C + 16,), jnp.int32),
            pltpu.VMEM((C, D), jnp.float32),
            pltpu.VMEM((C + 16,), jnp.int32),
            pltpu.VMEM((C, D), jnp.float32),
            pltpu.VMEM((W, D), jnp.float32),
            pltpu.VMEM((W, CNT_W), jnp.float32),
            pltpu.VMEM((16, D), jnp.float32),
            pltpu.VMEM((W,), jnp.int32),
            pltpu.VMEM((16,), jnp.int32),
            pltpu.VMEM_SHARED((S, D), jnp.float32),
            pltpu.VMEM_SHARED((S, D), jnp.float32),
            pltpu.SemaphoreType.DMA,
            pltpu.SemaphoreType.DMA,
        ],
    )
    def pool(x_hbm, b_hbm, out_hbm, cnt_hbm,
             idx0, xb0, idx1, xb1, win, cwin, stg, idxw, idxb,
             acc_sh, cnt_sh, sg0, sg1):
        cid = lax.axis_index("c")
        sid = lax.axis_index("s")
        wid = sid * 2 + cid
        k_first = wid * CPW
        iota16 = lax.iota(jnp.int32, 16)
        zvec = jnp.zeros((16,), jnp.float32)

        def gather(k, idx_v, xbuf, sem):
            base = jnp.where(k == NFULL, N - C, k * C)
            pltpu.async_copy(b_hbm.at[pl.ds(base, C)],
                             idx_v.at[pl.ds(0, C)], sem)
            pltpu.async_copy(x_hbm.at[pl.ds(base, C)], xbuf, sem)

        def wait_gather(idx_v, xbuf, sem):
            pltpu.make_async_copy(b_hbm.at[pl.ds(0, C)],
                                  idx_v.at[pl.ds(0, C)], sem).wait()
            pltpu.make_async_copy(x_hbm.at[pl.ds(0, C)], xbuf, sem).wait()

        @pl.when(k_first <= NFULL)
        def _():
            gather(k_first, idx0, xb0, sg0)

        def zero_window():
            def zr(i, carry):
                for u in range(8):
                    win[i, pl.ds(u * 16, 16)] = zvec
                cwin[i, :] = zvec
                return carry
            lax.fori_loop(0, W, zr, 0)

        zero_window()
        row0 = sid * RPT
        pltpu.sync_copy(win.at[pl.ds(0, RPT)], acc_sh.at[pl.ds(row0, RPT)])
        pltpu.sync_copy(win.at[pl.ds(0, RPT)], cnt_sh.at[pl.ds(row0, RPT)])
        plsc.subcore_barrier()

        def flush_window(w0):
            w0c = jnp.maximum(w0, 0)
            for q in range(8):
                idxw[pl.ds(q * 16, 16)] = w0c + q * 16 + iota16
            pltpu.sync_copy(win, acc_sh.at[idxw], add=True)
            for b in range(8):
                def fill(j, carry):
                    v = cwin[b * 16 + j, :]
                    for u in range(8):
                        stg[j, pl.ds(u * 16, 16)] = v
                    return carry
                lax.fori_loop(0, 16, fill, 0)
                idxb[...] = w0c + b * 16 + iota16
                pltpu.sync_copy(stg, cnt_sh.at[idxb], add=True)

        def compute(k, idx_v, xbuf, carry):
            lo = jnp.where(k == NFULL, C - REM, 0)
            hi = jnp.where(k <= NFULL, C, lo)

            def row(i, c):
                s_cur, w0, rlen, r = c
                s = idx_v[pl.ds(i, 16)][0]
                xrow = tuple(xbuf[i, pl.ds(u * 16, 16)] for u in range(8))
                is_new = s != s_cur

                @pl.when(is_new)
                def _flush_run():
                    off = s_cur - jnp.minimum(w0, s_cur)
                    for u in range(8):
                        sl = pl.ds(u * 16, 16)
                        win[off, sl] = win[off, sl] + r[u]
                    cwin[off, :] = cwin[off, :] + jnp.full((16,), 1.0) * rlen

                ovf = jnp.logical_and(w0 >= 0, s - w0 >= W)

                @pl.when(ovf)
                def _flush_win():
                    flush_window(w0)
                    zero_window()

                w2 = jnp.where(jnp.logical_or(w0 < 0, ovf), s, w0)
                r2 = tuple(jnp.where(is_new, xrow[u], r[u] + xrow[u])
                           for u in range(8))
                s2 = jnp.where(is_new, s, s_cur)
                rlen2 = jnp.where(is_new, 1.0, rlen + 1.0)
                return (s2, w2, rlen2, r2)

            return lax.fori_loop(lo, hi, row, carry)

        carry = (jnp.int32(0), jnp.int32(-1), jnp.float32(0.0),
                 tuple(zvec for _ in range(8)))

        def body(jj, carry):
            k0 = k_first + 2 * jj
            k1 = k0 + 1

            @pl.when(k0 <= NFULL)
            def _():
                wait_gather(idx0, xb0, sg0)

            @pl.when(k1 <= NFULL)
            def _():
                gather(k1, idx1, xb1, sg1)

            carry = compute(k0, idx0, xb0, carry)

            @pl.when(k0 + 2 <= NFULL)
            def _():
                gather(k0 + 2, idx0, xb0, sg0)

            @pl.when(k1 <= NFULL)
            def _():
                wait_gather(idx1, xb1, sg1)

            carry = compute(k1, idx1, xb1, carry)
            return carry

        carry = lax.fori_loop(0, (CPW - 1) // 2, body, carry)

        k_last = k_first + CPW - 1

        @pl.when(k_last <= NFULL)
        def _():
            wait_gather(idx0, xb0, sg0)

        carry = compute(k_last, idx0, xb0, carry)

        s_cur, w0, rlen, r = carry
        off = s_cur - jnp.minimum(jnp.maximum(w0, 0), s_cur)
        for u in range(8):
            sl = pl.ds(u * 16, 16)
            win[off, sl] = win[off, sl] + r[u]
        cwin[off, :] = cwin[off, :] + jnp.full((16,), 1.0) * rlen

        flush_window(w0)
        plsc.subcore_barrier()

        out_row = cid * S + row0
        pltpu.sync_copy(acc_sh.at[pl.ds(row0, RPT)], win.at[pl.ds(0, RPT)])
        pltpu.sync_copy(win.at[pl.ds(0, RPT)], out_hbm.at[pl.ds(out_row, RPT)])
        pltpu.sync_copy(cnt_sh.at[pl.ds(row0, RPT)], win.at[pl.ds(0, RPT)])
        pltpu.sync_copy(win.at[pl.ds(0, RPT)], cnt_hbm.at[pl.ds(out_row, RPT)])

    return pool(x, batch)


def _merge_body(p_ref, c_ref, o_ref):
    p = p_ref[0:S, :] + p_ref[S:2 * S, :]
    c = c_ref[0:S, 0:1] + c_ref[S:2 * S, 0:1]
    o_ref[...] = p / jnp.maximum(c, 1.0)


def kernel(x, batch):
    batch = batch.astype(jnp.int32)
    partial, cnt = _sc_pool(x, batch)
    out = pl.pallas_call(
        _merge_body,
        out_shape=jax.ShapeDtypeStruct((S, D), jnp.float32),
    )(partial, cnt)
    return out


# speedup vs baseline: 1.5662x; 1.5662x over previous
import functools

import jax
import jax.numpy as jnp
from jax import lax
from jax.experimental import pallas as pl
from jax.experimental.pallas import tpu as pltpu
from jax.experimental.pallas import tpu_sc as plsc

N = 100000
D = 128
S = 512
C = 128
NFULL = N // C
REM = N - NFULL * C
NW = 32
CPW = 25
RPT = S // 16
W = 128
CNT_W = 16


def _sc_pool(x, batch):
    mesh = plsc.VectorSubcoreMesh(core_axis_name="c", subcore_axis_name="s")

    @functools.partial(
        pl.kernel,
        mesh=mesh,
        out_type=[
            jax.ShapeDtypeStruct((2 * S, D), jnp.float32),
            jax.ShapeDtypeStruct((2 * S, D), jnp.float32),
        ],
        scratch_types=[
            pltpu.VMEM((C + 16,), jnp.int32),
            pltpu.VMEM((C, D), jnp.float32),
            pltpu.VMEM((C + 16,), jnp.int32),
            pltpu.VMEM((C, D), jnp.float32),
            pltpu.VMEM((W, D), jnp.float32),
            pltpu.VMEM((W, CNT_W), jnp.float32),
            pltpu.VMEM((16, D), jnp.float32),
            pltpu.VMEM((W,), jnp.int32),
            pltpu.VMEM((16,), jnp.int32),
            pltpu.VMEM_SHARED((S, D), jnp.float32),
            pltpu.VMEM_SHARED((S, D), jnp.float32),
            pltpu.SemaphoreType.DMA,
            pltpu.SemaphoreType.DMA,
        ],
    )
    def pool(x_hbm, b_hbm, out_hbm, cnt_hbm,
             idx0, xb0, idx1, xb1, win, cwin, stg, idxw, idxb,
             acc_sh, cnt_sh, sg0, sg1):
        cid = lax.axis_index("c")
        sid = lax.axis_index("s")
        wid = sid * 2 + cid
        k_first = wid * CPW
        iota16 = lax.iota(jnp.int32, 16)
        zvec = jnp.zeros((16,), jnp.float32)

        def gather(k, idx_v, xbuf, sem):
            base = jnp.where(k == NFULL, N - C, k * C)
            pltpu.async_copy(b_hbm.at[pl.ds(base, C)],
                             idx_v.at[pl.ds(0, C)], sem)
            pltpu.async_copy(x_hbm.at[pl.ds(base, C)], xbuf, sem)

        def wait_gather(idx_v, xbuf, sem):
            pltpu.make_async_copy(b_hbm.at[pl.ds(0, C)],
                                  idx_v.at[pl.ds(0, C)], sem).wait()
            pltpu.make_async_copy(x_hbm.at[pl.ds(0, C)], xbuf, sem).wait()

        @pl.when(k_first <= NFULL)
        def _():
            gather(k_first, idx0, xb0, sg0)

        def zero_window():
            def zr(i, carry):
                for u in range(8):
                    win[i, pl.ds(u * 16, 16)] = zvec
                cwin[i, :] = zvec
                return carry
            lax.fori_loop(0, W, zr, 0)

        zero_window()
        row0 = sid * RPT
        pltpu.sync_copy(win.at[pl.ds(0, RPT)], acc_sh.at[pl.ds(row0, RPT)])
        pltpu.sync_copy(win.at[pl.ds(0, RPT)], cnt_sh.at[pl.ds(row0, RPT)])
        plsc.subcore_barrier()

        def flush_window(w0):
            w0c = jnp.maximum(w0, 0)
            for q in range(8):
                idxw[pl.ds(q * 16, 16)] = w0c + q * 16 + iota16
            pltpu.sync_copy(win, acc_sh.at[idxw], add=True)
            for b in range(8):
                def fill(j, carry):
                    v = cwin[b * 16 + j, :]
                    for u in range(8):
                        stg[j, pl.ds(u * 16, 16)] = v
                    return carry
                lax.fori_loop(0, 16, fill, 0)
                idxb[...] = w0c + b * 16 + iota16
                pltpu.sync_copy(stg, cnt_sh.at[idxb], add=True)

        def compute(k, idx_v, xbuf, w0):
            lo_g = jnp.where(k == NFULL, (C - REM) // 16, 0)
            hi_g = jnp.where(k <= NFULL, C // 16, lo_g)

            def group(g, w0):
                r0 = g * 16
                ids = idx_v[pl.ds(r0, 16)]
                s0 = ids[0]
                s15 = ids[15]
                w1 = jnp.where(w0 < 0, s0, w0)
                ovf = s15 - w1 >= W

                @pl.when(ovf)
                def _():
                    flush_window(w1)
                    zero_window()

                w2 = jnp.where(ovf, s0, w1)

                @pl.when(s0 == s15)
                def _uniform():
                    off = s0 - w2
                    for u in range(8):
                        sl = pl.ds(u * 16, 16)
                        t01 = ((xbuf[r0, sl] + xbuf[r0 + 1, sl])
                               + (xbuf[r0 + 2, sl] + xbuf[r0 + 3, sl]))
                        t23 = ((xbuf[r0 + 4, sl] + xbuf[r0 + 5, sl])
                               + (xbuf[r0 + 6, sl] + xbuf[r0 + 7, sl]))
                        t45 = ((xbuf[r0 + 8, sl] + xbuf[r0 + 9, sl])
                               + (xbuf[r0 + 10, sl] + xbuf[r0 + 11, sl]))
                        t67 = ((xbuf[r0 + 12, sl] + xbuf[r0 + 13, sl])
                               + (xbuf[r0 + 14, sl] + xbuf[r0 + 15, sl]))
                        t = (t01 + t23) + (t45 + t67)
                        win[off, sl] = win[off, sl] + t
                    cwin[off, :] = cwin[off, :] + jnp.full((16,), 16.0)

                @pl.when(s0 != s15)
                def _mixed():
                    for ii in range(16):
                        off = ids[ii] - w2
                        for u in range(8):
                            sl = pl.ds(u * 16, 16)
                            win[off, sl] = win[off, sl] + xbuf[r0 + ii, sl]
                        cwin[off, :] = cwin[off, :] + jnp.full((16,), 1.0)

                return w2

            return lax.fori_loop(lo_g, hi_g, group, w0)

        carry = jnp.int32(-1)

        def body(jj, carry):
            k0 = k_first + 2 * jj
            k1 = k0 + 1

            @pl.when(k0 <= NFULL)
            def _():
                wait_gather(idx0, xb0, sg0)

            @pl.when(k1 <= NFULL)
            def _():
                gather(k1, idx1, xb1, sg1)

            carry = compute(k0, idx0, xb0, carry)

            @pl.when(k0 + 2 <= NFULL)
            def _():
                gather(k0 + 2, idx0, xb0, sg0)

            @pl.when(k1 <= NFULL)
            def _():
                wait_gather(idx1, xb1, sg1)

            carry = compute(k1, idx1, xb1, carry)
            return carry

        carry = lax.fori_loop(0, (CPW - 1) // 2, body, carry)

        k_last = k_first + CPW - 1

        @pl.when(k_last <= NFULL)
        def _():
            wait_gather(idx0, xb0, sg0)

        carry = compute(k_last, idx0, xb0, carry)

        flush_window(carry)
        plsc.subcore_barrier()

        out_row = cid * S + row0
        pltpu.sync_copy(acc_sh.at[pl.ds(row0, RPT)], win.at[pl.ds(0, RPT)])
        pltpu.sync_copy(win.at[pl.ds(0, RPT)], out_hbm.at[pl.ds(out_row, RPT)])
        pltpu.sync_copy(cnt_sh.at[pl.ds(row0, RPT)], win.at[pl.ds(0, RPT)])
        pltpu.sync_copy(win.at[pl.ds(0, RPT)], cnt_hbm.at[pl.ds(out_row, RPT)])

    return pool(x, batch)


def _merge_body(p_ref, c_ref, o_ref):
    p = p_ref[0:S, :] + p_ref[S:2 * S, :]
    c = c_ref[0:S, 0:1] + c_ref[S:2 * S, 0:1]
    o_ref[...] = p / jnp.maximum(c, 1.0)


def kernel(x, batch):
    batch = batch.astype(jnp.int32)
    partial, cnt = _sc_pool(x, batch)
    out = pl.pallas_call(
        _merge_body,
        out_shape=jax.ShapeDtypeStruct((S, D), jnp.float32),
    )(partial, cnt)
    return out
